# Initial kernel scaffold; baseline (speedup 1.0000x reference)
import jax
import jax.numpy as jnp
from jax.experimental import pallas as pl


def kernel(x, edge_index, edge_attr, W1a, b1a, g1, be1, W1b, b1b, W2a, b2a, g2, be2, W2b, b2b, W_ntn, V_ntn, b_ntn, W4, b4, W2l, b2l):
    raise NotImplementedError("write your pallas kernel here")


# speedup vs baseline: 8.1254x; 8.1254x over previous
import functools

import jax
import jax.numpy as jnp
from jax import lax
from jax.experimental import pallas as pl
from jax.experimental.pallas import tpu as pltpu
from jax.experimental.pallas import tpu_sc as plsc

N_NODES = 50000
N_EDGES = 1600000

C1 = 2000
R1 = N_EDGES // C1
R1W = R1 // 32
Q1 = C1 // 16

C2 = 125
A2 = 25
R2 = N_EDGES // (A2 * C2)
R2W = R2 // 16
NTILE = N_NODES // 16

NB = 1000
GRID = N_NODES // NB

_BN_INV = 1.0 / (1.0 + 1e-5) ** 0.5


def _sc_mesh():
    return plsc.VectorSubcoreMesh(core_axis_name="c", subcore_axis_name="s")


_SC_PARAMS = pltpu.CompilerParams(
    use_tc_tiling_on_sc=False, needs_layout_passes=False)


def _s1_body(xa_hbm, xb_hbm, src_hbm, dst_hbm, zn_hbm,
             p1_hbm, p2_hbm, xcol, acc, srcb, dstb):
    c = lax.axis_index("c")
    s = lax.axis_index("s")
    w = s * 2 + c

    def one_col(x_hbm, out_hbm):
        pltpu.sync_copy(x_hbm, xcol)
        pltpu.sync_copy(zn_hbm, acc)

        def row_body(g, carry):
            pltpu.sync_copy(src_hbm.at[w * R1W + g], srcb)
            pltpu.sync_copy(dst_hbm.at[w * R1W + g], dstb)

            def q_body(q, carry2):
                sv = srcb[pl.ds(q * 16, 16)]
                dv = dstb[pl.ds(q * 16, 16)]
                vals = plsc.load_gather(xcol, [sv])
                plsc.addupdate_scatter(acc, [dv], vals)
                return carry2

            lax.fori_loop(0, Q1, q_body, 0)
            return carry

        lax.fori_loop(0, R1W, row_body, 0)
        pltpu.sync_copy(acc, out_hbm.at[w])

    one_col(xa_hbm, p1_hbm)
    one_col(xb_hbm, p2_hbm)


def _s1_call(xa, xb, src2, dst2, zn):
    fn = functools.partial(
        pl.kernel,
        mesh=_sc_mesh(),
        out_type=(
            jax.ShapeDtypeStruct((32, N_NODES), jnp.float32),
            jax.ShapeDtypeStruct((32, N_NODES), jnp.float32),
        ),
        scratch_types=[
            pltpu.VMEM((N_NODES,), jnp.float32),
            pltpu.VMEM((N_NODES,), jnp.float32),
            pltpu.VMEM((C1,), jnp.int32),
            pltpu.VMEM((C1,), jnp.int32),
        ],
        compiler_params=_SC_PARAMS,
    )(_s1_body)
    return fn(xa, xb, src2, dst2, zn)


def _s2_body(h0, h1, h2, h3, h4, h5, h6, h7, src_hbm, dst_hbm, z2_hbm,
             a0, a1, a2, a3, a4, a5, a6, a7,
             srcb, dstb, rows, sem, acc):
    c = lax.axis_index("c")
    s = lax.axis_index("s")
    hrefs = (h0, h1, h2, h3, h4, h5, h6, h7)
    arefs = (a0, a1, a2, a3, a4, a5, a6, a7)
    base = s * NTILE

    for p in range(8):
        @pl.when(c == p // 4)
        def _(p=p):
            h_hbm = hrefs[p]
            out_hbm = arefs[p]
            pltpu.sync_copy(z2_hbm, acc.at[pl.ds(base, NTILE)])
            plsc.subcore_barrier()

            def g_body(g, carry):
                blk = s * R2W + g
                pltpu.sync_copy(src_hbm.at[blk], srcb)
                pltpu.sync_copy(dst_hbm.at[blk], dstb)

                def j_body(j, carry2):
                    pltpu.async_copy(h_hbm.at[srcb.at[j]], rows, sem).wait()
                    pltpu.sync_copy(rows, acc.at[dstb.at[j]], add=True)
                    return carry2

                lax.fori_loop(0, A2, j_body, 0)
                return carry

            lax.fori_loop(0, R2W, g_body, 0)
            plsc.subcore_barrier()
            pltpu.sync_copy(acc.at[pl.ds(base, NTILE)],
                            out_hbm.at[pl.ds(base, NTILE)])
            plsc.subcore_barrier()


def _s2_call(hs, src3, dst3, z2):
    out = tuple(jax.ShapeDtypeStruct((N_NODES, 32), jnp.float32)
                for _ in range(8))
    fn = functools.partial(
        pl.kernel,
        mesh=_sc_mesh(),
        out_type=out,
        scratch_types=[
            pltpu.VMEM((A2, C2), jnp.int32),
            pltpu.VMEM((A2, C2), jnp.int32),
            pltpu.VMEM((C2, 32), jnp.float32),
            pltpu.SemaphoreType.DMA,
            pltpu.VMEM_SHARED((N_NODES, 32), jnp.float32),
        ],
        compiler_params=_SC_PARAMS,
    )(_s2_body)
    return fn(*hs, src3, dst3, z2)


def _tca_body(x_ref, p1_ref, p2_ref, w1a_ref, b1a_ref, sc1_ref, be1_ref,
              w1b_ref, b1b_ref, *h_refs):
    s1 = x_ref[:, 1:2] + jnp.sum(p1_ref[...], axis=1, keepdims=True)
    s2 = x_ref[:, 0:1] + jnp.sum(p2_ref[...], axis=1, keepdims=True)

    def col(s):
        pre = s * w1a_ref[...] + b1a_ref[...]
        u = jnp.maximum(pre * sc1_ref[...] + be1_ref[...], 0.0)
        return jnp.dot(u, w1b_ref[...],
                       preferred_element_type=jnp.float32) + b1b_ref[...]

    h1 = col(s1)
    h2 = col(s2)
    for p in range(4):
        h_refs[p][...] = h1[:, 32 * p:32 * p + 32]
        h_refs[4 + p][...] = h2[:, 32 * p:32 * p + 32]


def _tca_call(x, p1t, p2t, w1a, b1a2, sc1, be12, w1b, b1b2):
    full = lambda shape: pl.BlockSpec(shape, lambda i: (0,) * len(shape))
    out_specs = [pl.BlockSpec((NB, 32), lambda i: (i, 0)) for _ in range(8)]
    return pl.pallas_call(
        _tca_body,
        grid=(GRID,),
        in_specs=[
            pl.BlockSpec((NB, 2), lambda i: (i, 0)),
            pl.BlockSpec((NB, 32), lambda i: (i, 0)),
            pl.BlockSpec((NB, 32), lambda i: (i, 0)),
            full((1, 256)), full((1, 256)), full((1, 256)), full((1, 256)),
            full((256, 128)), full((1, 128)),
        ],
        out_specs=out_specs,
        out_shape=[jax.ShapeDtypeStruct((N_NODES, 32), jnp.float32)
                   for _ in range(8)],
    )(x, p1t, p2t, w1a, b1a2, sc1, be12, w1b, b1b2)


def _tcb_body(*refs):
    h = refs[0:8]
    a = refs[8:16]
    (w2a_ref, b2a_ref, sc2_ref, be2_ref, w2b_ref, b2b_ref, vt_ref,
     wt2_ref, bsum_ref, btp_ref, ffp_ref) = refs[16:]

    def mlp2(parts):
        t = b2a_ref[...]
        for q in range(4):
            t = t + jnp.dot(parts[q], w2a_ref[32 * q:32 * q + 32, :],
                            preferred_element_type=jnp.float32)
        t = jnp.maximum(t * sc2_ref[...] + be2_ref[...], 0.0)
        return jnp.dot(t, w2b_ref[...],
                       preferred_element_type=jnp.float32) + b2b_ref[...]

    e1 = mlp2([h[q][...] + a[q][...] for q in range(4)])
    e2 = mlp2([h[4 + q][...] + a[4 + q][...] for q in range(4)])

    ffp_ref[...] = (
        jnp.dot(e1, vt_ref[0:64, :], preferred_element_type=jnp.float32)
        + jnp.dot(e2, vt_ref[64:128, :], preferred_element_type=jnp.float32))

    acc = jnp.zeros((NB, 64), jnp.float32) + bsum_ref[...]
    for j in range(64):
        acc = acc + e2[:, j:j + 1] * jnp.dot(
            e1, wt2_ref[j], preferred_element_type=jnp.float32)
    btp_ref[...] = acc


def _tcb_call(hs, aggs, w2a, b2a2, sc2, be22, w2b, b2b2, vt, wt2, bsum):
    full = lambda shape: pl.BlockSpec(shape, lambda i: (0,) * len(shape))
    nb32 = lambda: pl.BlockSpec((NB, 32), lambda i: (i, 0))
    return pl.pallas_call(
        _tcb_body,
        grid=(GRID,),
        in_specs=(
            [nb32() for _ in range(16)]
            + [full((128, 64)), full((1, 64)), full((1, 64)), full((1, 64)),
               full((64, 64)), full((1, 64)), full((128, 64)),
               full((64, 64, 64)), full((1, 1))]
        ),
        out_specs=[pl.BlockSpec((NB, 64), lambda i: (i, 0)),
                   pl.BlockSpec((NB, 64), lambda i: (i, 0))],
        out_shape=[jax.ShapeDtypeStruct((N_NODES, 64), jnp.float32),
                   jax.ShapeDtypeStruct((N_NODES, 64), jnp.float32)],
    )(*hs, *aggs, w2a, b2a2, sc2, be22, w2b, b2b2, vt, wt2, bsum)


def _tcc_body(btp_ref, ffp_ref, w4_ref, b4_ref, w2l_ref, b2l_ref, out_ref):
    o = jnp.tanh(btp_ref[...] + ffp_ref[...])
    z = jnp.maximum(
        jnp.dot(o, w4_ref[...], preferred_element_type=jnp.float32)
        + b4_ref[...], 0.0)
    z2 = jnp.dot(z, w2l_ref[...],
                 preferred_element_type=jnp.float32) + b2l_ref[...]
    m = jnp.max(z2, axis=1, keepdims=True)
    lse = m + jnp.log(jnp.sum(jnp.exp(z2 - m), axis=1, keepdims=True))
    out_ref[...] = z2 - lse


def _tcc_call(btp_r, ffp, w4, b42, w2l, b2l2):
    full = lambda shape: pl.BlockSpec(shape, lambda i: (0,) * len(shape))
    return pl.pallas_call(
        _tcc_body,
        grid=(GRID,),
        in_specs=[
            pl.BlockSpec((NB, 64), lambda i: (i, 0)),
            pl.BlockSpec((NB, 64), lambda i: (i, 0)),
            full((64, 6)), full((1, 6)), full((6, 6)), full((1, 6)),
        ],
        out_specs=pl.BlockSpec((NB, 6), lambda i: (i, 0)),
        out_shape=jax.ShapeDtypeStruct((N_NODES, 6), jnp.float32),
    )(btp_r, ffp, w4, b42, w2l, b2l2)


def kernel(x, edge_index, edge_attr, W1a, b1a, g1, be1, W1b, b1b,
           W2a, b2a, g2, be2, W2b, b2b, W_ntn, V_ntn, b_ntn, W4, b4,
           W2l, b2l):
    f32 = jnp.float32
    src = edge_index[0]
    dst = edge_index[1]
    xa = x[:, 1]
    xb = x[:, 0]
    src2 = src.reshape(R1, C1)
    dst2 = dst.reshape(R1, C1)
    src3 = src.reshape(R2, A2, C2)
    dst3 = dst.reshape(R2, A2, C2)
    zn = jnp.zeros((N_NODES,), f32)
    z2 = jnp.zeros((NTILE, 32), f32)

    p1, p2 = _s1_call(xa, xb, src2, dst2, zn)
    p1t = p1.T
    p2t = p2.T

    sc1 = (g1 * _BN_INV)[None, :]
    sc2 = (g2 * _BN_INV)[None, :]
    hs = _tca_call(x, p1t, p2t, W1a, b1a[None, :], sc1, be1[None, :],
                   W1b, b1b[None, :])

    aggs = _s2_call(hs, src3, dst3, z2)

    wt2 = jnp.transpose(W_ntn, (2, 1, 0))
    bsum = jnp.sum(b_ntn).reshape(1, 1)
    btpP, ffp = _tcb_call(hs, aggs, W2a, b2a[None, :], sc2, be2[None, :],
                          W2b, b2b[None, :], V_ntn, wt2, bsum)

    btp_r = jnp.transpose(btpP).reshape(N_NODES, 64)
    return _tcc_call(btp_r, ffp, W4, b4[None, :], W2l, b2l[None, :])
